# baseline (speedup 1.0000x reference)
import functools
import math

import jax
import jax.numpy as jnp
from jax import lax
from jax.experimental import pallas as pl
from jax.experimental.pallas import tpu as pltpu
from jax.experimental.pallas import tpu_sc as plsc

_NC, _NS, _LANES = 2, 16, 16
_NW = _NC * _NS
_NBUF = 5
_LOOKAHEAD = 2
_CHUNK = 128


def _build(n_rows: int, d: int):
    n_per_w = n_rows // _NW
    chunk = _CHUNK
    n_chunks = n_per_w // chunk
    assert n_chunks % _NBUF == 0 and n_per_w % chunk == 0
    scale = math.sqrt(d)
    mesh = plsc.VectorSubcoreMesh(core_axis_name="c", subcore_axis_name="s")

    @functools.partial(
        pl.kernel,
        mesh=mesh,
        out_type=jax.ShapeDtypeStruct((n_rows, d), jnp.float32),
        scratch_types=[
            pltpu.VMEM((n_per_w,), jnp.int32),
            pltpu.VMEM((_NBUF * chunk, d), jnp.float32),
            [pltpu.SemaphoreType.DMA] * _NBUF,
            [pltpu.SemaphoreType.DMA] * _NBUF,
        ],
    )
    def gather_scale(idx_hbm, table_hbm, out_hbm, idx_v, bufs, gsem, osem):
        wid = lax.axis_index("s") * _NC + lax.axis_index("c")
        base = wid * n_per_w
        pltpu.sync_copy(idx_hbm.at[pl.ds(base, n_per_w)], idx_v)

        def gather_descs(cid, b):
            off = cid * chunk
            buf0 = b * chunk
            descs = []
            p = 0
            while p < chunk:
                n = min(128, chunk - p)
                descs.append(
                    pltpu.make_async_copy(
                        table_hbm.at[idx_v.at[pl.ds(off + p, n)]],
                        bufs.at[pl.ds(buf0 + p, n)],
                        gsem[b],
                    )
                )
                p += n
            return descs

        def out_desc(cid, b):
            return pltpu.make_async_copy(
                bufs.at[pl.ds(b * chunk, chunk)],
                out_hbm.at[pl.ds(base + cid * chunk, chunk)],
                osem[b],
            )

        for c in range(_LOOKAHEAD):
            for g in gather_descs(c, c % _NBUF):
                g.start()

        @pl.loop(0, n_chunks, step=_NBUF)
        def _ring(ci):
            for b in range(_NBUF):
                cid = ci + b
                for g in gather_descs(cid, b):
                    g.wait()

                @pl.loop(0, chunk)
                def _row(r):
                    row = b * chunk + r
                    for j in range(d // _LANES):
                        sl = pl.ds(j * _LANES, _LANES)
                        bufs[row, sl] = bufs[row, sl] * scale

                out_desc(cid, b).start()

                nb_ = (b + _LOOKAHEAD) % _NBUF
                ncid = cid + _LOOKAHEAD

                @pl.when(jnp.logical_and(ncid >= _NBUF, ncid < n_chunks))
                def _():
                    out_desc(ncid - _NBUF, nb_).wait()

                @pl.when(ncid < n_chunks)
                def _():
                    for g in gather_descs(ncid, nb_):
                        g.start()

        for b in range(_NBUF):
            out_desc(n_chunks - _NBUF + b, b).wait()

    return gather_scale


def kernel(inputs, table):
    nb, l = inputs.shape
    v, d = table.shape
    idx = jnp.reshape(jnp.transpose(inputs), (l * nb,)).astype(jnp.int32)
    fn = _build(l * nb, d)
    out = fn(idx, table)
    return jnp.transpose(jnp.reshape(out, (l, nb, d)), (1, 0, 2))


# speedup vs baseline: 1.0115x; 1.0115x over previous
import functools
import math

import jax
import jax.numpy as jnp
from jax import lax
from jax.experimental import pallas as pl
from jax.experimental.pallas import tpu as pltpu
from jax.experimental.pallas import tpu_sc as plsc

_NC, _NS, _LANES = 2, 16, 16
_NW = _NC * _NS
_NBUF = 8
_LOOKAHEAD = 4
_CHUNK = 80


def _build(n_rows: int, d: int):
    n_per_w = n_rows // _NW
    chunk = _CHUNK
    n_chunks = n_per_w // chunk
    assert n_chunks % _NBUF == 0 and n_per_w % chunk == 0
    scale = math.sqrt(d)
    mesh = plsc.VectorSubcoreMesh(core_axis_name="c", subcore_axis_name="s")

    @functools.partial(
        pl.kernel,
        mesh=mesh,
        out_type=jax.ShapeDtypeStruct((n_rows, d), jnp.float32),
        scratch_types=[
            pltpu.VMEM((n_per_w,), jnp.int32),
            pltpu.VMEM((_NBUF * chunk, d), jnp.float32),
            [pltpu.SemaphoreType.DMA] * _NBUF,
            [pltpu.SemaphoreType.DMA] * _NBUF,
        ],
    )
    def gather_scale(idx_hbm, table_hbm, out_hbm, idx_v, bufs, gsem, osem):
        wid = lax.axis_index("s") * _NC + lax.axis_index("c")
        base = wid * n_per_w
        pltpu.sync_copy(idx_hbm.at[pl.ds(base, n_per_w)], idx_v)

        def gather_descs(cid, b):
            off = cid * chunk
            buf0 = b * chunk
            descs = []
            p = 0
            while p < chunk:
                n = min(128, chunk - p)
                descs.append(
                    pltpu.make_async_copy(
                        table_hbm.at[idx_v.at[pl.ds(off + p, n)]],
                        bufs.at[pl.ds(buf0 + p, n)],
                        gsem[b],
                    )
                )
                p += n
            return descs

        def out_desc(cid, b):
            return pltpu.make_async_copy(
                bufs.at[pl.ds(b * chunk, chunk)],
                out_hbm.at[pl.ds(base + cid * chunk, chunk)],
                osem[b],
            )

        for c in range(_LOOKAHEAD):
            for g in gather_descs(c, c % _NBUF):
                g.start()

        @pl.loop(0, n_chunks, step=_NBUF)
        def _ring(ci):
            for b in range(_NBUF):
                cid = ci + b
                for g in gather_descs(cid, b):
                    g.wait()

                @pl.loop(0, chunk)
                def _row(r):
                    row = b * chunk + r
                    for j in range(d // _LANES):
                        sl = pl.ds(j * _LANES, _LANES)
                        bufs[row, sl] = bufs[row, sl] * scale

                out_desc(cid, b).start()

                nb_ = (b + _LOOKAHEAD) % _NBUF
                ncid = cid + _LOOKAHEAD

                @pl.when(jnp.logical_and(ncid >= _NBUF, ncid < n_chunks))
                def _():
                    out_desc(ncid - _NBUF, nb_).wait()

                @pl.when(ncid < n_chunks)
                def _():
                    for g in gather_descs(ncid, nb_):
                        g.start()

        for b in range(_NBUF):
            out_desc(n_chunks - _NBUF + b, b).wait()

    return gather_scale


def kernel(inputs, table):
    nb, l = inputs.shape
    v, d = table.shape
    idx = jnp.reshape(jnp.transpose(inputs), (l * nb,)).astype(jnp.int32)
    fn = _build(l * nb, d)
    out = fn(idx, table)
    return jnp.transpose(jnp.reshape(out, (l, nb, d)), (1, 0, 2))
